# Initial kernel scaffold; baseline (speedup 1.0000x reference)
import jax
import jax.numpy as jnp
from jax.experimental import pallas as pl


def kernel(table, indices):
    raise NotImplementedError("write your pallas kernel here")


# speedup vs baseline: 1.5879x; 1.5879x over previous
import functools

import jax
import jax.numpy as jnp
from jax import lax
from jax.experimental import pallas as pl
from jax.experimental.pallas import tpu as pltpu
from jax.experimental.pallas import tpu_sc as plsc

NUM_EMBEDDINGS = 100
EMBED_DIM = 50
PADDED_DIM = 64
BATCH = 16384

NUM_CORES = 2
NUM_SUBCORES = 16
NUM_WORKERS = NUM_CORES * NUM_SUBCORES
ROWS_PER_WORKER = BATCH // NUM_WORKERS
CHUNK = 128
NUM_CHUNKS = ROWS_PER_WORKER // CHUNK


def kernel(table, indices):
    mesh = plsc.VectorSubcoreMesh(core_axis_name="c", subcore_axis_name="s")
    table_pad = jnp.pad(table, ((0, 0), (0, PADDED_DIM - EMBED_DIM)))
    idx3 = indices.reshape(NUM_WORKERS, NUM_CHUNKS, CHUNK)

    @functools.partial(
        pl.kernel,
        mesh=mesh,
        out_type=jax.ShapeDtypeStruct((BATCH, PADDED_DIM), jnp.float32),
        scratch_types=[
            pltpu.VMEM((NUM_CHUNKS, CHUNK), jnp.int32),
            pltpu.VMEM((NUM_CHUNKS, CHUNK, PADDED_DIM), jnp.float32),
            pltpu.SemaphoreType.DMA,
            pltpu.SemaphoreType.DMA,
        ],
        compiler_params=pltpu.CompilerParams(use_tc_tiling_on_sc=False),
    )
    def emb_kernel(table_hbm, idx_hbm, out_hbm, idx_v, rows_v, gsem, osem):
        wid = lax.axis_index("s") * NUM_CORES + lax.axis_index("c")
        base = wid * ROWS_PER_WORKER
        pltpu.sync_copy(idx_hbm.at[wid], idx_v)
        for c in range(NUM_CHUNKS):
            pltpu.async_copy(table_hbm.at[idx_v.at[c]], rows_v.at[c], gsem)
        for c in range(NUM_CHUNKS):
            pltpu.make_async_copy(table_hbm.at[idx_v.at[c]], rows_v.at[c], gsem).wait()
        for c in range(NUM_CHUNKS):
            pltpu.async_copy(
                rows_v.at[c], out_hbm.at[pl.ds(base + c * CHUNK, CHUNK)], osem)
        for c in range(NUM_CHUNKS):
            pltpu.make_async_copy(
                rows_v.at[c], out_hbm.at[pl.ds(base + c * CHUNK, CHUNK)], osem).wait()

    return emb_kernel(table_pad, idx3)[:, :EMBED_DIM]
